# Initial kernel scaffold; baseline (speedup 1.0000x reference)
import jax
import jax.numpy as jnp
from jax.experimental import pallas as pl


def kernel(X, A, W, b):
    raise NotImplementedError("write your pallas kernel here")


# speedup vs baseline: 3.3647x; 3.3647x over previous
import functools

import jax
import jax.numpy as jnp
from jax import lax
from jax.experimental import pallas as pl


def _score_body(w_ref, x_ref, out_ref):
    j = pl.program_id(1)

    @pl.when(j == 0)
    def _init():
        out_ref[...] = jnp.zeros_like(out_ref)

    w = w_ref[...][None, :]
    x = x_ref[0]
    out_ref[...] += jax.lax.dot_general(
        w, x, (((1,), (0,)), ((), ())),
        preferred_element_type=jnp.float32,
        precision=jax.lax.Precision.HIGHEST)[None]


def _select_body(logits_ref, a_ref, vals_ref, idx_ref, apool_ref, *, K):
    logits = logits_ref[...]
    B, V = logits.shape
    s = jax.nn.sigmoid(logits)

    su = s[:, :, None]
    sv = s[:, None, :]
    u_iota = lax.broadcasted_iota(jnp.int32, (B, V, V), 1)
    v_iota = lax.broadcasted_iota(jnp.int32, (B, V, V), 2)
    beats = (su > sv) | ((su == sv) & (u_iota < v_iota))
    rank = jnp.sum(beats.astype(jnp.float32), axis=1)
    keep = rank < K

    rank_i = rank.astype(jnp.int32)
    r_iota = lax.broadcasted_iota(jnp.int32, (B, V, K), 2)
    hit = rank_i[:, :, None] == r_iota
    vals_ref[...] = jnp.sum(jnp.where(hit, s[:, :, None], 0.0), axis=1)

    tri = (lax.broadcasted_iota(jnp.int32, (V, V), 0)
           <= lax.broadcasted_iota(jnp.int32, (V, V), 1)).astype(jnp.float32)
    keep_f = keep.astype(jnp.float32)
    slot = jax.lax.dot_general(
        keep_f, tri, (((1,), (0,)), ((), ())),
        preferred_element_type=jnp.float32).astype(jnp.int32) - 1
    k_iota = lax.broadcasted_iota(jnp.int32, (B, V, K), 2)
    sel = (rank_i[:, :, None] < K) & (slot[:, :, None] == k_iota)
    v_in_iota = lax.broadcasted_iota(jnp.int32, (B, V, K), 1)
    idx_ref[...] = jnp.sum(jnp.where(sel, v_in_iota, 0), axis=1)

    selT = (rank_i[:, None, :] < K) & (slot[:, None, :] == lax.broadcasted_iota(
        jnp.int32, (B, K, V), 1))
    sel_f = sel.astype(jnp.float32)
    selT_f = selT.astype(jnp.float32)
    for bi in range(B):
        a_b = a_ref[bi]
        rows = jax.lax.dot_general(
            selT_f[bi], a_b, (((1,), (0,)), ((), ())),
            preferred_element_type=jnp.float32,
            precision=jax.lax.Precision.HIGHEST)
        apool_ref[bi] = jax.lax.dot_general(
            rows, sel_f[bi], (((1,), (0,)), ((), ())),
            preferred_element_type=jnp.float32,
            precision=jax.lax.Precision.HIGHEST)


def _feature_body(idx_ref, vals_ref, x_ref, out_ref, *, K):
    x = x_ref[0]
    V = x.shape[-1]
    idx = idx_ref[0, 0]
    vals = vals_ref[0, 0]
    onehot = (lax.broadcasted_iota(jnp.int32, (V, K), 0) == idx[None, :])
    S = jnp.where(onehot, vals[None, :], 0.0)
    out_ref[0] = jax.lax.dot_general(
        x, S, (((1,), (0,)), ((), ())),
        preferred_element_type=jnp.float32,
        precision=jax.lax.Precision.HIGHEST)


def kernel(X, A, W, b):
    B, C, T, V = X.shape
    K = max(2, V // 2)
    CT = C * T
    RB = 1024
    NCT = CT // RB

    X2 = X.reshape(B, CT, V)
    w_ct = jnp.repeat(W[0] / T, T)

    logits = pl.pallas_call(
        _score_body,
        grid=(B, NCT),
        in_specs=[
            pl.BlockSpec((RB,), lambda i, j: (j,)),
            pl.BlockSpec((1, RB, V), lambda i, j: (i, j, 0)),
        ],
        out_specs=pl.BlockSpec((1, 1, V), lambda i, j: (i, 0, 0)),
        out_shape=jax.ShapeDtypeStruct((B, 1, V), jnp.float32),
    )(w_ct, X2)
    logits = logits.reshape(B, V) + b[0]

    vals, idx, a_pooled = pl.pallas_call(
        functools.partial(_select_body, K=K),
        out_shape=[
            jax.ShapeDtypeStruct((B, K), jnp.float32),
            jax.ShapeDtypeStruct((B, K), jnp.int32),
            jax.ShapeDtypeStruct((B, K, K), jnp.float32),
        ],
    )(logits, A)

    idx3 = idx.reshape(B, 1, K)
    vals3 = vals.reshape(B, 1, K)
    feats = pl.pallas_call(
        functools.partial(_feature_body, K=K),
        grid=(B, NCT),
        in_specs=[
            pl.BlockSpec((1, 1, K), lambda i, j: (i, 0, 0)),
            pl.BlockSpec((1, 1, K), lambda i, j: (i, 0, 0)),
            pl.BlockSpec((1, RB, V), lambda i, j: (i, j, 0)),
        ],
        out_specs=pl.BlockSpec((1, RB, K), lambda i, j: (i, j, 0)),
        out_shape=jax.ShapeDtypeStruct((B, CT, K), jnp.float32),
    )(idx3, vals3, X2)

    scaled_features = feats.reshape(B, C, T, K)
    return (a_pooled, scaled_features, idx)


# speedup vs baseline: 4.1722x; 1.2400x over previous
import functools

import jax
import jax.numpy as jnp
from jax import lax
from jax.experimental import pallas as pl
from jax.experimental.pallas import tpu as pltpu


def _score_body(w_ref, x_ref, out_ref, *, CB):
    j = pl.program_id(1)

    @pl.when(j == 0)
    def _init():
        out_ref[...] = jnp.zeros_like(out_ref)

    x = x_ref[0]
    acc = jnp.zeros((x.shape[2],), jnp.float32)
    for ci in range(CB):
        acc += jnp.sum(x[ci], axis=0) * w_ref[0, j * CB + ci]
    out_ref[...] += acc[None, None, :]


def _select_body(logits_ref, a_ref, vals_ref, idx_ref, apool_ref, *, K):
    logits = logits_ref[...]
    B, V = logits.shape
    s = jax.nn.sigmoid(logits)

    su = s[:, :, None]
    sv = s[:, None, :]
    u_iota = lax.broadcasted_iota(jnp.int32, (B, V, V), 1)
    v_iota = lax.broadcasted_iota(jnp.int32, (B, V, V), 2)
    beats = (su > sv) | ((su == sv) & (u_iota < v_iota))
    rank = jnp.sum(beats.astype(jnp.float32), axis=1)
    keep = rank < K

    rank_i = rank.astype(jnp.int32)
    r_iota = lax.broadcasted_iota(jnp.int32, (B, V, K), 2)
    hit = rank_i[:, :, None] == r_iota
    vals_ref[...] = jnp.sum(jnp.where(hit, s[:, :, None], 0.0), axis=1)

    tri = (lax.broadcasted_iota(jnp.int32, (V, V), 0)
           <= lax.broadcasted_iota(jnp.int32, (V, V), 1)).astype(jnp.float32)
    keep_f = keep.astype(jnp.float32)
    slot = jax.lax.dot_general(
        keep_f, tri, (((1,), (0,)), ((), ())),
        preferred_element_type=jnp.float32).astype(jnp.int32) - 1
    k_iota = lax.broadcasted_iota(jnp.int32, (B, V, K), 2)
    sel = (rank_i[:, :, None] < K) & (slot[:, :, None] == k_iota)
    v_in_iota = lax.broadcasted_iota(jnp.int32, (B, V, K), 1)
    idx_ref[...] = jnp.sum(jnp.where(sel, v_in_iota, 0), axis=1)

    selT = (rank_i[:, None, :] < K) & (slot[:, None, :] == lax.broadcasted_iota(
        jnp.int32, (B, K, V), 1))
    sel_f = sel.astype(jnp.float32)
    selT_f = selT.astype(jnp.float32)
    for bi in range(B):
        a_b = a_ref[bi]
        rows = jax.lax.dot_general(
            selT_f[bi], a_b, (((1,), (0,)), ((), ())),
            preferred_element_type=jnp.float32,
            precision=jax.lax.Precision.HIGHEST)
        apool_ref[bi] = jax.lax.dot_general(
            rows, sel_f[bi], (((1,), (0,)), ((), ())),
            preferred_element_type=jnp.float32,
            precision=jax.lax.Precision.HIGHEST)


def _feature_body(idx_ref, vals_ref, x_ref, out_ref, *, K):
    x = x_ref[0]
    V = x.shape[-1]
    idx = idx_ref[0, 0]
    vals = vals_ref[0, 0]
    onehot = (lax.broadcasted_iota(jnp.int32, (V, K), 0) == idx[None, :])
    S = jnp.where(onehot, vals[None, :], 0.0)
    out_ref[0] = jax.lax.dot_general(
        x, S, (((1,), (0,)), ((), ())),
        preferred_element_type=jnp.float32,
        precision=jax.lax.Precision.HIGHEST)


def kernel(X, A, W, b):
    B, C, T, V = X.shape
    K = max(2, V // 2)
    CT = C * T
    RB = 1024
    NCT = CT // RB

    X2 = X.reshape(B, CT, V)
    w_sc = W / T
    CB = 32
    NC = C // CB

    logits = pl.pallas_call(
        functools.partial(_score_body, CB=CB),
        grid=(B, NC),
        in_specs=[
            pl.BlockSpec(memory_space=pltpu.SMEM),
            pl.BlockSpec((1, CB, T, V), lambda i, j: (i, j, 0, 0)),
        ],
        out_specs=pl.BlockSpec((1, 1, V), lambda i, j: (i, 0, 0)),
        out_shape=jax.ShapeDtypeStruct((B, 1, V), jnp.float32),
    )(w_sc, X)
    logits = logits.reshape(B, V) + b[0]

    vals, idx, a_pooled = pl.pallas_call(
        functools.partial(_select_body, K=K),
        out_shape=[
            jax.ShapeDtypeStruct((B, K), jnp.float32),
            jax.ShapeDtypeStruct((B, K), jnp.int32),
            jax.ShapeDtypeStruct((B, K, K), jnp.float32),
        ],
    )(logits, A)

    idx3 = idx.reshape(B, 1, K)
    vals3 = vals.reshape(B, 1, K)
    feats = pl.pallas_call(
        functools.partial(_feature_body, K=K),
        grid=(B, NCT),
        in_specs=[
            pl.BlockSpec((1, 1, K), lambda i, j: (i, 0, 0)),
            pl.BlockSpec((1, 1, K), lambda i, j: (i, 0, 0)),
            pl.BlockSpec((1, RB, V), lambda i, j: (i, j, 0)),
        ],
        out_specs=pl.BlockSpec((1, RB, K), lambda i, j: (i, j, 0)),
        out_shape=jax.ShapeDtypeStruct((B, CT, K), jnp.float32),
    )(idx3, vals3, X2)

    scaled_features = feats.reshape(B, C, T, K)
    return (a_pooled, scaled_features, idx)
